# Initial kernel scaffold; baseline (speedup 1.0000x reference)
import jax
import jax.numpy as jnp
from jax.experimental import pallas as pl


def kernel(x, edge_index, W1, b1, W2, b2, Wl, bl):
    raise NotImplementedError("write your pallas kernel here")


# speedup vs baseline: 10.2544x; 10.2544x over previous
import functools

import jax
import jax.numpy as jnp
from jax import lax
from jax.experimental import pallas as pl
from jax.experimental.pallas import tpu as pltpu
from jax.experimental.pallas import tpu_sc as plsc

N_NODES = 10000
N_EDGES = 160000
D_IN = 256
D_HID = 512

_MESH = plsc.VectorSubcoreMesh(core_axis_name="c", subcore_axis_name="s")

_EB = 128


@functools.partial(
    pl.kernel,
    out_type=jax.ShapeDtypeStruct((2, N_NODES), jnp.float32),
    mesh=_MESH,
    scratch_types=[
        pltpu.VMEM((_EB,), jnp.int32),
        pltpu.VMEM((_EB,), jnp.float32),
        pltpu.VMEM_SHARED((N_NODES,), jnp.float32),
    ],
)
def _deg_kernel(dst_hbm, zeros_hbm, out_hbm, idx_v, ones_v, acc):
    c = lax.axis_index("c")
    s = lax.axis_index("s")

    @pl.when(s == 0)
    def _init():
        pltpu.sync_copy(zeros_hbm, acc)

    for i in range(_EB // 16):
        ones_v[pl.ds(i * 16, 16)] = jnp.ones((16,), jnp.float32)
    plsc.subcore_barrier()

    half = N_EDGES // 2
    per_tile = 5120
    base = c * half + s * per_tile
    nb = jnp.where(s == 15, 25, per_tile // _EB)

    def body(b, carry):
        pltpu.sync_copy(dst_hbm.at[pl.ds(base + b * _EB, _EB)], idx_v)
        pltpu.sync_copy(ones_v, acc.at[idx_v], add=True)
        return carry

    lax.fori_loop(0, nb, body, 0)
    plsc.subcore_barrier()

    @pl.when(s == 0)
    def _writeback():
        pltpu.sync_copy(acc, out_hbm.at[c])


def _make_agg(n_chunks):
    @functools.partial(
        pl.kernel,
        out_type=jax.ShapeDtypeStruct((n_chunks * N_NODES, 128), jnp.float32),
        mesh=_MESH,
        scratch_types=[
            pltpu.VMEM((_EB,), jnp.int32),
            pltpu.VMEM((_EB,), jnp.int32),
            pltpu.VMEM((_EB, 128), jnp.float32),
            pltpu.VMEM_SHARED((N_NODES, 128), jnp.float32),
        ],
    )
    def _agg(src_hbm, dst_hbm, g_hbm, out_hbm, src_v, dst_v, rows_v, acc):
        c = lax.axis_index("c")
        s = lax.axis_index("s")
        per_tile = 10240
        base = s * per_tile
        nb = jnp.where(s == 15, 50, per_tile // _EB)

        for jj in range(n_chunks // 2):
            chunk = c + 2 * jj
            row_off = chunk * N_NODES

            @pl.when(s == 0)
            def _init():
                pltpu.sync_copy(g_hbm.at[pl.ds(row_off, N_NODES)], acc)

            plsc.subcore_barrier()

            def body(b, carry):
                e0 = base + b * _EB
                pltpu.sync_copy(src_hbm.at[pl.ds(e0, _EB)], src_v)
                pltpu.sync_copy(dst_hbm.at[pl.ds(e0, _EB)], dst_v)
                for i in range(_EB // 16):
                    sl = pl.ds(i * 16, 16)
                    src_v[sl] = src_v[sl] + row_off
                pltpu.sync_copy(g_hbm.at[src_v], rows_v)
                pltpu.sync_copy(rows_v, acc.at[dst_v], add=True)
                return carry

            lax.fori_loop(0, nb, body, 0)
            plsc.subcore_barrier()

            @pl.when(s == 0)
            def _writeback():
                pltpu.sync_copy(acc, out_hbm.at[pl.ds(row_off, N_NODES)])

            plsc.subcore_barrier()

    return _agg


_agg2 = _make_agg(2)
_agg4 = _make_agg(4)


_BN = 1000
_NB = N_NODES // _BN


def _dinv(deg_ref):
    return lax.rsqrt(deg_ref[:, 0:1] + deg_ref[:, 1:2] + 1.0)


def _scale_body(x_ref, deg_ref, out_ref):
    g = x_ref[...] * _dinv(deg_ref)
    out_ref[0] = g[:, :128]
    out_ref[1] = g[:, 128:]


def _mm1_body(a_ref, deg_ref, w1_ref, b1_ref, out_ref):
    p = jnp.dot(a_ref[0], w1_ref[0:128, :], preferred_element_type=jnp.float32)
    p += jnp.dot(a_ref[1], w1_ref[128:256, :], preferred_element_type=jnp.float32)
    di = _dinv(deg_ref)
    h = jnp.maximum(di * p + b1_ref[...], 0.0)
    g = di * h
    for j in range(4):
        out_ref[j] = g[:, j * 128:(j + 1) * 128]


def _mm2_body(a_ref, deg_ref, w2_ref, b2_ref, wl_ref, bl_ref, out_ref):
    i = pl.program_id(0)
    p = jnp.dot(a_ref[0], w2_ref[0:128, :], preferred_element_type=jnp.float32)
    for j in range(1, 4):
        p += jnp.dot(a_ref[j], w2_ref[j * 128:(j + 1) * 128, :],
                     preferred_element_type=jnp.float32)
    h = jnp.maximum(_dinv(deg_ref) * p + b2_ref[...], 0.0)
    part = jnp.sum(h * wl_ref[...], axis=(0, 1), keepdims=True)

    @pl.when(i == 0)
    def _first():
        out_ref[...] = part

    @pl.when(i != 0)
    def _accum():
        out_ref[...] = out_ref[...] + part

    @pl.when(i == _NB - 1)
    def _final():
        v = out_ref[...] * (1.0 / N_NODES) + bl_ref[...]
        out_ref[...] = jax.nn.sigmoid(v)


_scale = pl.pallas_call(
    _scale_body,
    grid=(_NB,),
    in_specs=[
        pl.BlockSpec((_BN, D_IN), lambda i: (i, 0)),
        pl.BlockSpec((_BN, 2), lambda i: (i, 0)),
    ],
    out_specs=pl.BlockSpec((2, _BN, 128), lambda i: (0, i, 0)),
    out_shape=jax.ShapeDtypeStruct((2, N_NODES, 128), jnp.float32),
)

_mm1 = pl.pallas_call(
    _mm1_body,
    grid=(_NB,),
    in_specs=[
        pl.BlockSpec((2, _BN, 128), lambda i: (0, i, 0)),
        pl.BlockSpec((_BN, 2), lambda i: (i, 0)),
        pl.BlockSpec((D_IN, D_HID), lambda i: (0, 0)),
        pl.BlockSpec((1, D_HID), lambda i: (0, 0)),
    ],
    out_specs=pl.BlockSpec((4, _BN, 128), lambda i: (0, i, 0)),
    out_shape=jax.ShapeDtypeStruct((4, N_NODES, 128), jnp.float32),
)

_mm2 = pl.pallas_call(
    _mm2_body,
    grid=(_NB,),
    in_specs=[
        pl.BlockSpec((4, _BN, 128), lambda i: (0, i, 0)),
        pl.BlockSpec((_BN, 2), lambda i: (i, 0)),
        pl.BlockSpec((D_HID, D_HID), lambda i: (0, 0)),
        pl.BlockSpec((1, D_HID), lambda i: (0, 0)),
        pl.BlockSpec((1, D_HID), lambda i: (0, 0)),
        pl.BlockSpec((1, 1), lambda i: (0, 0)),
    ],
    out_specs=pl.BlockSpec((1, 1), lambda i: (0, 0)),
    out_shape=jax.ShapeDtypeStruct((1, 1), jnp.float32),
)


def kernel(x, edge_index, W1, b1, W2, b2, Wl, bl):
    src = edge_index[0]
    dst = edge_index[1]
    zeros_n = jnp.zeros((N_NODES,), jnp.float32)

    deg_parts = _deg_kernel(dst, zeros_n)
    deg_t = deg_parts.T

    g0 = _scale(x, deg_t)
    acc0 = _agg2(src, dst, g0.reshape(2 * N_NODES, 128))

    g1 = _mm1(acc0.reshape(2, N_NODES, 128), deg_t, W1,
              b1.reshape(1, D_HID))
    acc1 = _agg4(src, dst, g1.reshape(4 * N_NODES, 128))

    out = _mm2(acc1.reshape(4, N_NODES, 128), deg_t, W2,
               b2.reshape(1, D_HID), Wl.reshape(1, D_HID),
               bl.reshape(1, 1))
    return out.reshape(1)


# speedup vs baseline: 17.5951x; 1.7159x over previous
import functools

import jax
import jax.numpy as jnp
from jax import lax
from jax.experimental import pallas as pl
from jax.experimental.pallas import tpu as pltpu
from jax.experimental.pallas import tpu_sc as plsc

N_NODES = 10000
N_EDGES = 160000
D_IN = 256
D_HID = 512

_MESH = plsc.VectorSubcoreMesh(core_axis_name="c", subcore_axis_name="s")

_EB = 128


@functools.partial(
    pl.kernel,
    out_type=jax.ShapeDtypeStruct((2, N_NODES), jnp.float32),
    mesh=_MESH,
    scratch_types=[
        pltpu.VMEM((_EB,), jnp.int32),
        pltpu.VMEM((_EB,), jnp.float32),
        pltpu.VMEM_SHARED((N_NODES,), jnp.float32),
    ],
)
def _deg_kernel(dst_hbm, zeros_hbm, out_hbm, idx_v, ones_v, acc):
    c = lax.axis_index("c")
    s = lax.axis_index("s")

    @pl.when(s == 0)
    def _init():
        pltpu.sync_copy(zeros_hbm, acc)

    for i in range(_EB // 16):
        ones_v[pl.ds(i * 16, 16)] = jnp.ones((16,), jnp.float32)
    plsc.subcore_barrier()

    half = N_EDGES // 2
    per_tile = 5120
    base = c * half + s * per_tile
    nb = jnp.where(s == 15, 25, per_tile // _EB)

    def body(b, carry):
        pltpu.sync_copy(dst_hbm.at[pl.ds(base + b * _EB, _EB)], idx_v)
        pltpu.sync_copy(ones_v, acc.at[idx_v], add=True)
        return carry

    lax.fori_loop(0, nb, body, 0)
    plsc.subcore_barrier()

    @pl.when(s == 0)
    def _writeback():
        pltpu.sync_copy(acc, out_hbm.at[c])


_RPT = 80
_NPT = 632


def _make_agg(n_chunks):
    @functools.partial(
        pl.kernel,
        out_type=jax.ShapeDtypeStruct((n_chunks * N_NODES, 128), jnp.float32),
        mesh=_MESH,
        scratch_types=[
            pltpu.VMEM((_RPT, 1, _EB), jnp.int32),
            pltpu.VMEM((_EB,), jnp.int32),
            pltpu.VMEM((_EB,), jnp.int32),
            pltpu.VMEM((_EB, 128), jnp.float32),
            pltpu.VMEM((_EB, 128), jnp.float32),
            pltpu.VMEM_SHARED((N_NODES, 128), jnp.float32),
            pltpu.SemaphoreType.DMA,
            pltpu.SemaphoreType.DMA,
            pltpu.SemaphoreType.DMA,
        ],
    )
    def _agg(src_hbm, dst_hbm, g_hbm, out_hbm,
             dst_all, src_v0, src_v1, buf0, buf1, acc, sem_i, sem_g, sem_s):
        c = lax.axis_index("c")
        s = lax.axis_index("s")
        nb = jnp.where(s == 15, 50, _RPT)
        row0 = s * _RPT
        e0 = s * (_RPT * _EB)

        @pl.when(s < 15)
        def _stage():
            pltpu.sync_copy(dst_hbm.at[pl.ds(row0, _RPT)], dst_all)

        @pl.when(s == 15)
        def _stage_last():
            pltpu.sync_copy(dst_hbm.at[pl.ds(row0, 50)],
                            dst_all.at[pl.ds(0, 50)])

        def fire_idx(b, src_v):
            pltpu.async_copy(src_hbm.at[pl.ds(e0 + b * _EB, _EB)], src_v,
                             sem_i)

        def wait_idx(src_v):
            pltpu.make_async_copy(src_hbm.at[pl.ds(0, _EB)], src_v,
                                  sem_i).wait()

        def adjust(src_v, row_off):
            for i in range(_EB // 16):
                sl = pl.ds(i * 16, 16)
                src_v[sl] = src_v[sl] + row_off

        def fire_gather(src_v, buf):
            pltpu.async_copy(g_hbm.at[src_v], buf, sem_g)

        def wait_gather(buf):
            pltpu.make_async_copy(g_hbm.at[pl.ds(0, _EB)], buf, sem_g).wait()

        def fire_scatter(b, buf):
            pltpu.async_copy(buf, acc.at[dst_all.at[b, 0]], sem_s, add=True)

        def wait_scatter(buf):
            pltpu.make_async_copy(buf, acc.at[pl.ds(0, _EB)], sem_s).wait()

        for jj in range(n_chunks // 2):
            chunk = c + 2 * jj
            row_off = chunk * N_NODES

            @pl.when(s < 15)
            def _init():
                pltpu.sync_copy(g_hbm.at[pl.ds(row_off + s * _NPT, _NPT)],
                                acc.at[pl.ds(s * _NPT, _NPT)])

            @pl.when(s == 15)
            def _init_last():
                pltpu.sync_copy(g_hbm.at[pl.ds(row_off + 15 * _NPT, 520)],
                                acc.at[pl.ds(15 * _NPT, 520)])

            pltpu.sync_copy(src_hbm.at[pl.ds(e0, _EB)], src_v0)
            adjust(src_v0, row_off)
            fire_gather(src_v0, buf0)
            fire_idx(1, src_v1)
            plsc.subcore_barrier()

            def body(w, carry):
                b0 = 2 * w
                b1 = b0 + 1
                wait_gather(buf0)

                @pl.when(w > 0)
                def _():
                    wait_scatter(buf1)

                wait_idx(src_v1)
                adjust(src_v1, row_off)
                fire_gather(src_v1, buf1)

                @pl.when(b0 + 2 < nb)
                def _():
                    fire_idx(b0 + 2, src_v0)

                fire_scatter(b0, buf0)
                wait_gather(buf1)
                wait_scatter(buf0)

                @pl.when(b1 + 1 < nb)
                def _():
                    wait_idx(src_v0)
                    adjust(src_v0, row_off)
                    fire_gather(src_v0, buf0)

                @pl.when(b1 + 2 < nb)
                def _():
                    fire_idx(b1 + 2, src_v1)

                fire_scatter(b1, buf1)
                return carry

            lax.fori_loop(0, nb // 2, body, 0)
            wait_scatter(buf1)
            plsc.subcore_barrier()

            @pl.when(s < 15)
            def _writeback():
                pltpu.sync_copy(acc.at[pl.ds(s * _NPT, _NPT)],
                                out_hbm.at[pl.ds(row_off + s * _NPT, _NPT)])

            @pl.when(s == 15)
            def _writeback_last():
                pltpu.sync_copy(acc.at[pl.ds(15 * _NPT, 520)],
                                out_hbm.at[pl.ds(row_off + 15 * _NPT, 520)])

            plsc.subcore_barrier()

    return _agg


_agg2 = _make_agg(2)
_agg4 = _make_agg(4)


_BN = 1000
_NB = N_NODES // _BN


def _dinv(deg_ref):
    return lax.rsqrt(deg_ref[:, 0:1] + deg_ref[:, 1:2] + 1.0)


def _scale_body(x_ref, deg_ref, out_ref):
    g = x_ref[...] * _dinv(deg_ref)
    out_ref[0] = g[:, :128]
    out_ref[1] = g[:, 128:]


def _mm1_body(a_ref, deg_ref, w1_ref, b1_ref, out_ref):
    p = jnp.dot(a_ref[0], w1_ref[0:128, :], preferred_element_type=jnp.float32)
    p += jnp.dot(a_ref[1], w1_ref[128:256, :], preferred_element_type=jnp.float32)
    di = _dinv(deg_ref)
    h = jnp.maximum(di * p + b1_ref[...], 0.0)
    g = di * h
    for j in range(4):
        out_ref[j] = g[:, j * 128:(j + 1) * 128]


def _mm2_body(a_ref, deg_ref, w2_ref, b2_ref, wl_ref, bl_ref, out_ref):
    i = pl.program_id(0)
    p = jnp.dot(a_ref[0], w2_ref[0:128, :], preferred_element_type=jnp.float32)
    for j in range(1, 4):
        p += jnp.dot(a_ref[j], w2_ref[j * 128:(j + 1) * 128, :],
                     preferred_element_type=jnp.float32)
    h = jnp.maximum(_dinv(deg_ref) * p + b2_ref[...], 0.0)
    part = jnp.sum(h * wl_ref[...], axis=(0, 1), keepdims=True)

    @pl.when(i == 0)
    def _first():
        out_ref[...] = part

    @pl.when(i != 0)
    def _accum():
        out_ref[...] = out_ref[...] + part

    @pl.when(i == _NB - 1)
    def _final():
        v = out_ref[...] * (1.0 / N_NODES) + bl_ref[...]
        out_ref[...] = jax.nn.sigmoid(v)


_scale = pl.pallas_call(
    _scale_body,
    grid=(_NB,),
    in_specs=[
        pl.BlockSpec((_BN, D_IN), lambda i: (i, 0)),
        pl.BlockSpec((_BN, 2), lambda i: (i, 0)),
    ],
    out_specs=pl.BlockSpec((2, _BN, 128), lambda i: (0, i, 0)),
    out_shape=jax.ShapeDtypeStruct((2, N_NODES, 128), jnp.float32),
)

_mm1 = pl.pallas_call(
    _mm1_body,
    grid=(_NB,),
    in_specs=[
        pl.BlockSpec((2, _BN, 128), lambda i: (0, i, 0)),
        pl.BlockSpec((_BN, 2), lambda i: (i, 0)),
        pl.BlockSpec((D_IN, D_HID), lambda i: (0, 0)),
        pl.BlockSpec((1, D_HID), lambda i: (0, 0)),
    ],
    out_specs=pl.BlockSpec((4, _BN, 128), lambda i: (0, i, 0)),
    out_shape=jax.ShapeDtypeStruct((4, N_NODES, 128), jnp.float32),
)

_mm2 = pl.pallas_call(
    _mm2_body,
    grid=(_NB,),
    in_specs=[
        pl.BlockSpec((4, _BN, 128), lambda i: (0, i, 0)),
        pl.BlockSpec((_BN, 2), lambda i: (i, 0)),
        pl.BlockSpec((D_HID, D_HID), lambda i: (0, 0)),
        pl.BlockSpec((1, D_HID), lambda i: (0, 0)),
        pl.BlockSpec((1, D_HID), lambda i: (0, 0)),
        pl.BlockSpec((1, 1), lambda i: (0, 0)),
    ],
    out_specs=pl.BlockSpec((1, 1), lambda i: (0, 0)),
    out_shape=jax.ShapeDtypeStruct((1, 1), jnp.float32),
)


def kernel(x, edge_index, W1, b1, W2, b2, Wl, bl):
    src = edge_index[0]
    dst = edge_index[1]
    dst3d = dst.reshape(N_EDGES // _EB, 1, _EB)
    zeros_n = jnp.zeros((N_NODES,), jnp.float32)

    deg_parts = _deg_kernel(dst, zeros_n)
    deg_t = deg_parts.T

    g0 = _scale(x, deg_t)
    acc0 = _agg2(src, dst3d, g0.reshape(2 * N_NODES, 128))

    g1 = _mm1(acc0.reshape(2, N_NODES, 128), deg_t, W1,
              b1.reshape(1, D_HID))
    acc1 = _agg4(src, dst3d, g1.reshape(4 * N_NODES, 128))

    out = _mm2(acc1.reshape(4, N_NODES, 128), deg_t, W2,
               b2.reshape(1, D_HID), Wl.reshape(1, D_HID),
               bl.reshape(1, 1))
    return out.reshape(1)


# speedup vs baseline: 17.7433x; 1.0084x over previous
import functools

import jax
import jax.numpy as jnp
from jax import lax
from jax.experimental import pallas as pl
from jax.experimental.pallas import tpu as pltpu
from jax.experimental.pallas import tpu_sc as plsc

N_NODES = 10000
N_EDGES = 160000
D_IN = 256
D_HID = 512

_MESH = plsc.VectorSubcoreMesh(core_axis_name="c", subcore_axis_name="s")

_EB = 128


@functools.partial(
    pl.kernel,
    out_type=jax.ShapeDtypeStruct((2, N_NODES), jnp.float32),
    mesh=_MESH,
    scratch_types=[
        pltpu.VMEM((_EB,), jnp.int32),
        pltpu.VMEM((_EB,), jnp.float32),
        pltpu.VMEM_SHARED((N_NODES,), jnp.float32),
    ],
)
def _deg_kernel(dst_hbm, zeros_hbm, out_hbm, idx_v, ones_v, acc):
    c = lax.axis_index("c")
    s = lax.axis_index("s")

    @pl.when(s == 0)
    def _init():
        pltpu.sync_copy(zeros_hbm, acc)

    for i in range(_EB // 16):
        ones_v[pl.ds(i * 16, 16)] = jnp.ones((16,), jnp.float32)
    plsc.subcore_barrier()

    half = N_EDGES // 2
    per_tile = 5120
    base = c * half + s * per_tile
    nb = jnp.where(s == 15, 25, per_tile // _EB)

    def body(b, carry):
        pltpu.sync_copy(dst_hbm.at[pl.ds(base + b * _EB, _EB)], idx_v)
        pltpu.sync_copy(ones_v, acc.at[idx_v], add=True)
        return carry

    lax.fori_loop(0, nb, body, 0)
    plsc.subcore_barrier()

    @pl.when(s == 0)
    def _writeback():
        pltpu.sync_copy(acc, out_hbm.at[c])


_RPT = 80
_NPT = 640


def _make_agg(n_chunks):
    @functools.partial(
        pl.kernel,
        out_type=jax.ShapeDtypeStruct((n_chunks * N_NODES, 128), jnp.float32),
        mesh=_MESH,
        scratch_types=[
            pltpu.VMEM((_RPT, 1, _EB), jnp.int32),
            pltpu.VMEM((_EB,), jnp.int32),
            pltpu.VMEM((_EB,), jnp.int32),
            pltpu.VMEM((_EB, 128), jnp.float32),
            pltpu.VMEM((_EB, 128), jnp.float32),
            pltpu.VMEM_SHARED((N_NODES, 128), jnp.float32),
            pltpu.SemaphoreType.DMA,
            pltpu.SemaphoreType.DMA,
            pltpu.SemaphoreType.DMA,
        ],
    )
    def _agg(src_hbm, dst_hbm, g_hbm, out_hbm,
             dst_all, src_v0, src_v1, buf0, buf1, acc, sem_i, sem_g, sem_s):
        c = lax.axis_index("c")
        s = lax.axis_index("s")
        nb = jnp.where(s == 15, 50, _RPT)
        row0 = s * _RPT
        e0 = s * (_RPT * _EB)

        @pl.when(s < 15)
        def _stage():
            pltpu.sync_copy(dst_hbm.at[pl.ds(row0, _RPT)], dst_all)

        @pl.when(s == 15)
        def _stage_last():
            pltpu.sync_copy(dst_hbm.at[pl.ds(row0, 50)],
                            dst_all.at[pl.ds(0, 50)])

        def fire_idx(b, src_v):
            pltpu.async_copy(src_hbm.at[pl.ds(e0 + b * _EB, _EB)], src_v,
                             sem_i)

        def wait_idx(src_v):
            pltpu.make_async_copy(src_hbm.at[pl.ds(0, _EB)], src_v,
                                  sem_i).wait()

        def adjust(src_v, row_off):
            for i in range(_EB // 16):
                sl = pl.ds(i * 16, 16)
                src_v[sl] = src_v[sl] + row_off

        def fire_gather(src_v, buf):
            pltpu.async_copy(g_hbm.at[src_v], buf, sem_g)

        def wait_gather(buf):
            pltpu.make_async_copy(g_hbm.at[pl.ds(0, _EB)], buf, sem_g).wait()

        def fire_scatter(b, buf):
            pltpu.async_copy(buf, acc.at[dst_all.at[b, 0]], sem_s, add=True)

        def wait_scatter(buf):
            pltpu.make_async_copy(buf, acc.at[pl.ds(0, _EB)], sem_s).wait()

        for jj in range(n_chunks // 2):
            chunk = c + 2 * jj
            row_off = chunk * N_NODES

            @pl.when(s < 15)
            def _init():
                pltpu.sync_copy(g_hbm.at[pl.ds(row_off + s * _NPT, _NPT)],
                                acc.at[pl.ds(s * _NPT, _NPT)])

            @pl.when(s == 15)
            def _init_last():
                pltpu.sync_copy(g_hbm.at[pl.ds(row_off + 15 * _NPT, 400)],
                                acc.at[pl.ds(15 * _NPT, 400)])

            pltpu.sync_copy(src_hbm.at[pl.ds(e0, _EB)], src_v0)
            adjust(src_v0, row_off)
            fire_gather(src_v0, buf0)
            fire_idx(1, src_v1)
            plsc.subcore_barrier()

            def body(w, carry):
                b0 = 2 * w
                b1 = b0 + 1
                wait_gather(buf0)

                @pl.when(w > 0)
                def _():
                    wait_scatter(buf1)

                wait_idx(src_v1)
                adjust(src_v1, row_off)
                fire_gather(src_v1, buf1)

                @pl.when(b0 + 2 < nb)
                def _():
                    fire_idx(b0 + 2, src_v0)

                fire_scatter(b0, buf0)
                wait_gather(buf1)
                wait_scatter(buf0)

                @pl.when(b1 + 1 < nb)
                def _():
                    wait_idx(src_v0)
                    adjust(src_v0, row_off)
                    fire_gather(src_v0, buf0)

                @pl.when(b1 + 2 < nb)
                def _():
                    fire_idx(b1 + 2, src_v1)

                fire_scatter(b1, buf1)
                return carry

            lax.fori_loop(0, nb // 2, body, 0)
            wait_scatter(buf1)
            plsc.subcore_barrier()

            @pl.when(s < 15)
            def _writeback():
                pltpu.sync_copy(acc.at[pl.ds(s * _NPT, _NPT)],
                                out_hbm.at[pl.ds(row_off + s * _NPT, _NPT)])

            @pl.when(s == 15)
            def _writeback_last():
                pltpu.sync_copy(acc.at[pl.ds(15 * _NPT, 400)],
                                out_hbm.at[pl.ds(row_off + 15 * _NPT, 400)])

            plsc.subcore_barrier()

    return _agg


_agg2 = _make_agg(2)
_agg4 = _make_agg(4)


_BN = 2000
_NB = N_NODES // _BN


def _dinv(deg_ref):
    return lax.rsqrt(deg_ref[:, 0:1] + deg_ref[:, 1:2] + 1.0)


def _scale_body(x_ref, deg_ref, out_ref):
    g = x_ref[...] * _dinv(deg_ref)
    out_ref[0] = g[:, :128]
    out_ref[1] = g[:, 128:]


def _mm1_body(a_ref, deg_ref, w1_ref, b1_ref, out_ref):
    p = jnp.dot(a_ref[0], w1_ref[0:128, :], preferred_element_type=jnp.float32)
    p += jnp.dot(a_ref[1], w1_ref[128:256, :], preferred_element_type=jnp.float32)
    di = _dinv(deg_ref)
    h = jnp.maximum(di * p + b1_ref[...], 0.0)
    g = di * h
    for j in range(4):
        out_ref[j] = g[:, j * 128:(j + 1) * 128]


def _mm2_body(a_ref, deg_ref, w2_ref, b2_ref, wl_ref, bl_ref, out_ref):
    i = pl.program_id(0)
    p = jnp.dot(a_ref[0], w2_ref[0:128, :], preferred_element_type=jnp.float32)
    for j in range(1, 4):
        p += jnp.dot(a_ref[j], w2_ref[j * 128:(j + 1) * 128, :],
                     preferred_element_type=jnp.float32)
    h = jnp.maximum(_dinv(deg_ref) * p + b2_ref[...], 0.0)
    part = jnp.sum(h * wl_ref[...], axis=(0, 1), keepdims=True)

    @pl.when(i == 0)
    def _first():
        out_ref[...] = part

    @pl.when(i != 0)
    def _accum():
        out_ref[...] = out_ref[...] + part

    @pl.when(i == _NB - 1)
    def _final():
        v = out_ref[...] * (1.0 / N_NODES) + bl_ref[...]
        out_ref[...] = jax.nn.sigmoid(v)


_scale = pl.pallas_call(
    _scale_body,
    grid=(_NB,),
    in_specs=[
        pl.BlockSpec((_BN, D_IN), lambda i: (i, 0)),
        pl.BlockSpec((_BN, 2), lambda i: (i, 0)),
    ],
    out_specs=pl.BlockSpec((2, _BN, 128), lambda i: (0, i, 0)),
    out_shape=jax.ShapeDtypeStruct((2, N_NODES, 128), jnp.float32),
)

_mm1 = pl.pallas_call(
    _mm1_body,
    grid=(_NB,),
    in_specs=[
        pl.BlockSpec((2, _BN, 128), lambda i: (0, i, 0)),
        pl.BlockSpec((_BN, 2), lambda i: (i, 0)),
        pl.BlockSpec((D_IN, D_HID), lambda i: (0, 0)),
        pl.BlockSpec((1, D_HID), lambda i: (0, 0)),
    ],
    out_specs=pl.BlockSpec((4, _BN, 128), lambda i: (0, i, 0)),
    out_shape=jax.ShapeDtypeStruct((4, N_NODES, 128), jnp.float32),
)

_mm2 = pl.pallas_call(
    _mm2_body,
    grid=(_NB,),
    in_specs=[
        pl.BlockSpec((4, _BN, 128), lambda i: (0, i, 0)),
        pl.BlockSpec((_BN, 2), lambda i: (i, 0)),
        pl.BlockSpec((D_HID, D_HID), lambda i: (0, 0)),
        pl.BlockSpec((1, D_HID), lambda i: (0, 0)),
        pl.BlockSpec((1, D_HID), lambda i: (0, 0)),
        pl.BlockSpec((1, 1), lambda i: (0, 0)),
    ],
    out_specs=pl.BlockSpec((1, 1), lambda i: (0, 0)),
    out_shape=jax.ShapeDtypeStruct((1, 1), jnp.float32),
)


def kernel(x, edge_index, W1, b1, W2, b2, Wl, bl):
    src = edge_index[0]
    dst = edge_index[1]
    dst3d = dst.reshape(N_EDGES // _EB, 1, _EB)
    zeros_n = jnp.zeros((N_NODES,), jnp.float32)

    deg_parts = _deg_kernel(dst, zeros_n)
    deg_t = deg_parts.T

    g0 = _scale(x, deg_t)
    acc0 = _agg2(src, dst3d, g0.reshape(2 * N_NODES, 128))

    g1 = _mm1(acc0.reshape(2, N_NODES, 128), deg_t, W1,
              b1.reshape(1, D_HID))
    acc1 = _agg4(src, dst3d, g1.reshape(4 * N_NODES, 128))

    out = _mm2(acc1.reshape(4, N_NODES, 128), deg_t, W2,
               b2.reshape(1, D_HID), Wl.reshape(1, D_HID),
               bl.reshape(1, 1))
    return out.reshape(1)


# speedup vs baseline: 21.7438x; 1.2255x over previous
import functools

import jax
import jax.numpy as jnp
from jax import lax
from jax.experimental import pallas as pl
from jax.experimental.pallas import tpu as pltpu
from jax.experimental.pallas import tpu_sc as plsc

N_NODES = 10000
N_EDGES = 160000
D_IN = 256
D_HID = 512

_MESH = plsc.VectorSubcoreMesh(core_axis_name="c", subcore_axis_name="s")

_EB = 128


@functools.partial(
    pl.kernel,
    out_type=jax.ShapeDtypeStruct((2, N_NODES), jnp.float32),
    mesh=_MESH,
    scratch_types=[
        pltpu.VMEM((_EB,), jnp.int32),
        pltpu.VMEM((_EB,), jnp.float32),
        pltpu.VMEM_SHARED((N_NODES,), jnp.float32),
    ],
)
def _deg_kernel(dst_hbm, zeros_hbm, out_hbm, idx_v, ones_v, acc):
    c = lax.axis_index("c")
    s = lax.axis_index("s")

    @pl.when(s == 0)
    def _init():
        pltpu.sync_copy(zeros_hbm, acc)

    for i in range(_EB // 16):
        ones_v[pl.ds(i * 16, 16)] = jnp.ones((16,), jnp.float32)
    plsc.subcore_barrier()

    half = N_EDGES // 2
    per_tile = 5120
    base = c * half + s * per_tile
    nb = jnp.where(s == 15, 25, per_tile // _EB)

    def body(b, carry):
        pltpu.sync_copy(dst_hbm.at[pl.ds(base + b * _EB, _EB)], idx_v)
        pltpu.sync_copy(ones_v, acc.at[idx_v], add=True)
        return carry

    lax.fori_loop(0, nb, body, 0)
    plsc.subcore_barrier()

    @pl.when(s == 0)
    def _writeback():
        pltpu.sync_copy(acc, out_hbm.at[c])


_RPT = 80
_NPT = 640


def _make_agg(n_chunks):
    @functools.partial(
        pl.kernel,
        out_type=jax.ShapeDtypeStruct((n_chunks * N_NODES, 128), jnp.float32),
        mesh=_MESH,
        scratch_types=[
            [pltpu.VMEM((_EB,), jnp.int32) for _ in range(3)],
            [pltpu.VMEM((_EB,), jnp.int32) for _ in range(3)],
            [pltpu.VMEM((_EB, 128), jnp.float32) for _ in range(3)],
            pltpu.VMEM_SHARED((N_NODES, 128), jnp.float32),
            [pltpu.SemaphoreType.DMA for _ in range(3)],
            [pltpu.SemaphoreType.DMA for _ in range(3)],
            [pltpu.SemaphoreType.DMA for _ in range(3)],
            [pltpu.SemaphoreType.DMA for _ in range(3)],
        ],
    )
    def _agg(src_hbm, dst_hbm, g_hbm, out_hbm,
             src_v, dst_v, bufs, acc, sem_si, sem_di, sem_g, sem_s):
        c = lax.axis_index("c")
        s = lax.axis_index("s")
        nb = jnp.where(s == 15, 50, _RPT)
        e0 = s * (_RPT * _EB)

        def f_sidx(b, j):
            pltpu.async_copy(src_hbm.at[pl.ds(e0 + b * _EB, _EB)], src_v[j],
                             sem_si[j])

        def w_sidx(j):
            pltpu.make_async_copy(src_hbm.at[pl.ds(0, _EB)], src_v[j],
                                  sem_si[j]).wait()

        def f_didx(b, j):
            pltpu.async_copy(dst_hbm.at[pl.ds(e0 + b * _EB, _EB)], dst_v[j],
                             sem_di[j])

        def w_didx(j):
            pltpu.make_async_copy(dst_hbm.at[pl.ds(0, _EB)], dst_v[j],
                                  sem_di[j]).wait()

        def adjust(j, row_off):
            for i in range(_EB // 16):
                sl = pl.ds(i * 16, 16)
                src_v[j][sl] = src_v[j][sl] + row_off

        def f_g(j):
            pltpu.async_copy(g_hbm.at[src_v[j]], bufs[j], sem_g[j])

        def w_g(j):
            pltpu.make_async_copy(g_hbm.at[pl.ds(0, _EB)], bufs[j],
                                  sem_g[j]).wait()

        def f_s(j):
            pltpu.async_copy(bufs[j], acc.at[dst_v[j]], sem_s[j], add=True)

        def w_s(j):
            pltpu.make_async_copy(bufs[j], acc.at[pl.ds(0, _EB)],
                                  sem_s[j]).wait()

        for jj in range(n_chunks // 2):
            chunk = c + 2 * jj
            row_off = chunk * N_NODES

            @pl.when(s < 15)
            def _init():
                pltpu.sync_copy(g_hbm.at[pl.ds(row_off + s * _NPT, _NPT)],
                                acc.at[pl.ds(s * _NPT, _NPT)])

            @pl.when(s == 15)
            def _init_last():
                pltpu.sync_copy(g_hbm.at[pl.ds(row_off + 15 * _NPT, 400)],
                                acc.at[pl.ds(15 * _NPT, 400)])

            pltpu.sync_copy(src_hbm.at[pl.ds(e0, _EB)], src_v[0])
            adjust(0, row_off)
            f_g(0)
            pltpu.sync_copy(src_hbm.at[pl.ds(e0 + _EB, _EB)], src_v[1])
            adjust(1, row_off)
            f_g(1)
            f_sidx(2, 2)
            f_didx(0, 0)
            f_didx(1, 1)
            plsc.subcore_barrier()

            def body(w, carry):
                for j in range(3):
                    b = 3 * w + j

                    @pl.when(b < nb)
                    def _batch():
                        j2 = (j + 2) % 3
                        w_g(j)

                        @pl.when(b + 3 < nb)
                        def _():
                            f_sidx(b + 3, j)

                        w_didx(j)
                        f_s(j)

                        @pl.when(b + 2 < nb)
                        def _():
                            @pl.when(b >= 1)
                            def _():
                                w_s(j2)

                            w_sidx(j2)
                            adjust(j2, row_off)
                            f_g(j2)
                            f_didx(b + 2, j2)

                return carry

            nw = (nb + 2) // 3
            lax.fori_loop(0, nw, body, 0)
            w_s(2)
            w_s(0)
            w_s(1)
            plsc.subcore_barrier()

            @pl.when(s < 15)
            def _writeback():
                pltpu.sync_copy(acc.at[pl.ds(s * _NPT, _NPT)],
                                out_hbm.at[pl.ds(row_off + s * _NPT, _NPT)])

            @pl.when(s == 15)
            def _writeback_last():
                pltpu.sync_copy(acc.at[pl.ds(15 * _NPT, 400)],
                                out_hbm.at[pl.ds(row_off + 15 * _NPT, 400)])

            plsc.subcore_barrier()

    return _agg


_agg2 = _make_agg(2)
_agg4 = _make_agg(4)


_BN = 2000
_NB = N_NODES // _BN


def _dinv(deg_ref):
    return lax.rsqrt(deg_ref[:, 0:1] + deg_ref[:, 1:2] + 1.0)


def _scale_body(x_ref, deg_ref, out_ref):
    g = x_ref[...] * _dinv(deg_ref)
    out_ref[0] = g[:, :128]
    out_ref[1] = g[:, 128:]


def _mm1_body(a_ref, deg_ref, w1_ref, b1_ref, out_ref):
    p = jnp.dot(a_ref[0], w1_ref[0:128, :], preferred_element_type=jnp.float32)
    p += jnp.dot(a_ref[1], w1_ref[128:256, :], preferred_element_type=jnp.float32)
    di = _dinv(deg_ref)
    h = jnp.maximum(di * p + b1_ref[...], 0.0)
    g = di * h
    for j in range(4):
        out_ref[j] = g[:, j * 128:(j + 1) * 128]


def _mm2_body(a_ref, deg_ref, w2_ref, b2_ref, wl_ref, bl_ref, out_ref):
    i = pl.program_id(0)
    p = jnp.dot(a_ref[0], w2_ref[0:128, :], preferred_element_type=jnp.float32)
    for j in range(1, 4):
        p += jnp.dot(a_ref[j], w2_ref[j * 128:(j + 1) * 128, :],
                     preferred_element_type=jnp.float32)
    h = jnp.maximum(_dinv(deg_ref) * p + b2_ref[...], 0.0)
    part = jnp.sum(h * wl_ref[...], axis=(0, 1), keepdims=True)

    @pl.when(i == 0)
    def _first():
        out_ref[...] = part

    @pl.when(i != 0)
    def _accum():
        out_ref[...] = out_ref[...] + part

    @pl.when(i == _NB - 1)
    def _final():
        v = out_ref[...] * (1.0 / N_NODES) + bl_ref[...]
        out_ref[...] = jax.nn.sigmoid(v)


_scale = pl.pallas_call(
    _scale_body,
    grid=(_NB,),
    in_specs=[
        pl.BlockSpec((_BN, D_IN), lambda i: (i, 0)),
        pl.BlockSpec((_BN, 2), lambda i: (i, 0)),
    ],
    out_specs=pl.BlockSpec((2, _BN, 128), lambda i: (0, i, 0)),
    out_shape=jax.ShapeDtypeStruct((2, N_NODES, 128), jnp.float32),
)

_mm1 = pl.pallas_call(
    _mm1_body,
    grid=(_NB,),
    in_specs=[
        pl.BlockSpec((2, _BN, 128), lambda i: (0, i, 0)),
        pl.BlockSpec((_BN, 2), lambda i: (i, 0)),
        pl.BlockSpec((D_IN, D_HID), lambda i: (0, 0)),
        pl.BlockSpec((1, D_HID), lambda i: (0, 0)),
    ],
    out_specs=pl.BlockSpec((4, _BN, 128), lambda i: (0, i, 0)),
    out_shape=jax.ShapeDtypeStruct((4, N_NODES, 128), jnp.float32),
)

_mm2 = pl.pallas_call(
    _mm2_body,
    grid=(_NB,),
    in_specs=[
        pl.BlockSpec((4, _BN, 128), lambda i: (0, i, 0)),
        pl.BlockSpec((_BN, 2), lambda i: (i, 0)),
        pl.BlockSpec((D_HID, D_HID), lambda i: (0, 0)),
        pl.BlockSpec((1, D_HID), lambda i: (0, 0)),
        pl.BlockSpec((1, D_HID), lambda i: (0, 0)),
        pl.BlockSpec((1, 1), lambda i: (0, 0)),
    ],
    out_specs=pl.BlockSpec((1, 1), lambda i: (0, 0)),
    out_shape=jax.ShapeDtypeStruct((1, 1), jnp.float32),
)


def kernel(x, edge_index, W1, b1, W2, b2, Wl, bl):
    src = edge_index[0]
    dst = edge_index[1]
    zeros_n = jnp.zeros((N_NODES,), jnp.float32)

    deg_parts = _deg_kernel(dst, zeros_n)
    deg_t = deg_parts.T

    g0 = _scale(x, deg_t)
    acc0 = _agg2(src, dst, g0.reshape(2 * N_NODES, 128))

    g1 = _mm1(acc0.reshape(2, N_NODES, 128), deg_t, W1,
              b1.reshape(1, D_HID))
    acc1 = _agg4(src, dst, g1.reshape(4 * N_NODES, 128))

    out = _mm2(acc1.reshape(4, N_NODES, 128), deg_t, W2,
               b2.reshape(1, D_HID), Wl.reshape(1, D_HID),
               bl.reshape(1, 1))
    return out.reshape(1)


# speedup vs baseline: 22.4949x; 1.0345x over previous
import functools

import jax
import jax.numpy as jnp
from jax import lax
from jax.experimental import pallas as pl
from jax.experimental.pallas import tpu as pltpu
from jax.experimental.pallas import tpu_sc as plsc

N_NODES = 10000
N_EDGES = 160000
D_IN = 256
D_HID = 512

_MESH = plsc.VectorSubcoreMesh(core_axis_name="c", subcore_axis_name="s")

_EB = 128


@functools.partial(
    pl.kernel,
    out_type=jax.ShapeDtypeStruct((2, N_NODES), jnp.float32),
    mesh=_MESH,
    scratch_types=[
        [pltpu.VMEM((_EB,), jnp.int32) for _ in range(3)],
        pltpu.VMEM((_EB,), jnp.float32),
        pltpu.VMEM_SHARED((N_NODES,), jnp.float32),
        [pltpu.SemaphoreType.DMA for _ in range(3)],
        [pltpu.SemaphoreType.DMA for _ in range(3)],
    ],
)
def _deg_kernel(dst_hbm, zeros_hbm, out_hbm, idx_v, ones_v, acc,
                sem_i, sem_s):
    c = lax.axis_index("c")
    s = lax.axis_index("s")

    @pl.when(s == 0)
    def _init():
        pltpu.sync_copy(zeros_hbm, acc)

    for i in range(_EB // 16):
        ones_v[pl.ds(i * 16, 16)] = jnp.ones((16,), jnp.float32)

    half = N_EDGES // 2
    per_tile = 5120
    base = c * half + s * per_tile
    nb = jnp.where(s == 15, 25, per_tile // _EB)

    def f_idx(b, j):
        pltpu.async_copy(dst_hbm.at[pl.ds(base + b * _EB, _EB)], idx_v[j],
                         sem_i[j])

    def w_idx(j):
        pltpu.make_async_copy(dst_hbm.at[pl.ds(0, _EB)], idx_v[j],
                              sem_i[j]).wait()

    def f_s(j):
        pltpu.async_copy(ones_v, acc.at[idx_v[j]], sem_s[j], add=True)

    def w_s(j):
        pltpu.make_async_copy(ones_v, acc.at[pl.ds(0, _EB)], sem_s[j]).wait()

    f_idx(0, 0)
    f_idx(1, 1)
    plsc.subcore_barrier()

    def body(w, carry):
        for j in range(3):
            b = 3 * w + j

            @pl.when(b < nb)
            def _batch():
                j2 = (j + 2) % 3
                w_idx(j)
                f_s(j)

                @pl.when(b + 2 < nb)
                def _():
                    @pl.when(b >= 1)
                    def _():
                        w_s(j2)

                    f_idx(b + 2, j2)

        return carry

    lax.fori_loop(0, (nb + 2) // 3, body, 0)
    w_s(2)
    w_s(0)
    w_s(1)
    plsc.subcore_barrier()

    @pl.when(s == 0)
    def _writeback():
        pltpu.sync_copy(acc, out_hbm.at[c])


_RPT = 80
_NPT = 640


def _make_agg(n_chunks):
    @functools.partial(
        pl.kernel,
        out_type=jax.ShapeDtypeStruct((n_chunks * N_NODES, 128), jnp.float32),
        mesh=_MESH,
        scratch_types=[
            [pltpu.VMEM((_EB,), jnp.int32) for _ in range(3)],
            [pltpu.VMEM((_EB,), jnp.int32) for _ in range(3)],
            [pltpu.VMEM((_EB, 128), jnp.float32) for _ in range(3)],
            pltpu.VMEM_SHARED((N_NODES, 128), jnp.float32),
            [pltpu.SemaphoreType.DMA for _ in range(3)],
            [pltpu.SemaphoreType.DMA for _ in range(3)],
            [pltpu.SemaphoreType.DMA for _ in range(3)],
            [pltpu.SemaphoreType.DMA for _ in range(3)],
        ],
    )
    def _agg(src_hbm, dst_hbm, g_hbm, out_hbm,
             src_v, dst_v, bufs, acc, sem_si, sem_di, sem_g, sem_s):
        c = lax.axis_index("c")
        s = lax.axis_index("s")
        nb = jnp.where(s == 15, 50, _RPT)
        e0 = s * (_RPT * _EB)

        def f_sidx(b, j):
            pltpu.async_copy(src_hbm.at[pl.ds(e0 + b * _EB, _EB)], src_v[j],
                             sem_si[j])

        def w_sidx(j):
            pltpu.make_async_copy(src_hbm.at[pl.ds(0, _EB)], src_v[j],
                                  sem_si[j]).wait()

        def f_didx(b, j):
            pltpu.async_copy(dst_hbm.at[pl.ds(e0 + b * _EB, _EB)], dst_v[j],
                             sem_di[j])

        def w_didx(j):
            pltpu.make_async_copy(dst_hbm.at[pl.ds(0, _EB)], dst_v[j],
                                  sem_di[j]).wait()

        def adjust(j, row_off):
            for i in range(_EB // 16):
                sl = pl.ds(i * 16, 16)
                src_v[j][sl] = src_v[j][sl] + row_off

        def f_g(j):
            pltpu.async_copy(g_hbm.at[src_v[j]], bufs[j], sem_g[j])

        def w_g(j):
            pltpu.make_async_copy(g_hbm.at[pl.ds(0, _EB)], bufs[j],
                                  sem_g[j]).wait()

        def f_s(j):
            pltpu.async_copy(bufs[j], acc.at[dst_v[j]], sem_s[j], add=True)

        def w_s(j):
            pltpu.make_async_copy(bufs[j], acc.at[pl.ds(0, _EB)],
                                  sem_s[j]).wait()

        for jj in range(n_chunks // 2):
            chunk = c + 2 * jj
            row_off = chunk * N_NODES

            @pl.when(s < 15)
            def _init():
                pltpu.sync_copy(g_hbm.at[pl.ds(row_off + s * _NPT, _NPT)],
                                acc.at[pl.ds(s * _NPT, _NPT)])

            @pl.when(s == 15)
            def _init_last():
                pltpu.sync_copy(g_hbm.at[pl.ds(row_off + 15 * _NPT, 400)],
                                acc.at[pl.ds(15 * _NPT, 400)])

            pltpu.sync_copy(src_hbm.at[pl.ds(e0, _EB)], src_v[0])
            adjust(0, row_off)
            f_g(0)
            pltpu.sync_copy(src_hbm.at[pl.ds(e0 + _EB, _EB)], src_v[1])
            adjust(1, row_off)
            f_g(1)
            f_sidx(2, 2)
            f_didx(0, 0)
            f_didx(1, 1)
            plsc.subcore_barrier()

            def body(w, carry):
                for j in range(3):
                    b = 3 * w + j

                    @pl.when(b < nb)
                    def _batch():
                        j2 = (j + 2) % 3
                        w_g(j)

                        @pl.when(b + 3 < nb)
                        def _():
                            f_sidx(b + 3, j)

                        w_didx(j)
                        f_s(j)

                        @pl.when(b + 2 < nb)
                        def _():
                            @pl.when(b >= 1)
                            def _():
                                w_s(j2)

                            w_sidx(j2)
                            adjust(j2, row_off)
                            f_g(j2)
                            f_didx(b + 2, j2)

                return carry

            nw = (nb + 2) // 3
            lax.fori_loop(0, nw, body, 0)
            w_s(2)
            w_s(0)
            w_s(1)
            plsc.subcore_barrier()

            @pl.when(s < 15)
            def _writeback():
                pltpu.sync_copy(acc.at[pl.ds(s * _NPT, _NPT)],
                                out_hbm.at[pl.ds(row_off + s * _NPT, _NPT)])

            @pl.when(s == 15)
            def _writeback_last():
                pltpu.sync_copy(acc.at[pl.ds(15 * _NPT, 400)],
                                out_hbm.at[pl.ds(row_off + 15 * _NPT, 400)])

            plsc.subcore_barrier()

    return _agg


_agg2 = _make_agg(2)
_agg4 = _make_agg(4)


_BN = 2000
_NB = N_NODES // _BN


def _dinv(deg_ref):
    return lax.rsqrt(deg_ref[:, 0:1] + deg_ref[:, 1:2] + 1.0)


def _scale_body(x_ref, deg_ref, out_ref):
    g = x_ref[...] * _dinv(deg_ref)
    out_ref[0] = g[:, :128]
    out_ref[1] = g[:, 128:]


def _mm1_body(a_ref, deg_ref, w1_ref, b1_ref, out_ref):
    p = jnp.dot(a_ref[0].astype(jnp.bfloat16), w1_ref[0:128, :],
                preferred_element_type=jnp.float32)
    p += jnp.dot(a_ref[1].astype(jnp.bfloat16), w1_ref[128:256, :],
                 preferred_element_type=jnp.float32)
    di = _dinv(deg_ref)
    h = jnp.maximum(di * p + b1_ref[...], 0.0)
    g = di * h
    for j in range(4):
        out_ref[j] = g[:, j * 128:(j + 1) * 128]


def _mm2_body(a_ref, deg_ref, w2_ref, b2_ref, wl_ref, bl_ref, out_ref):
    i = pl.program_id(0)
    p = jnp.dot(a_ref[0].astype(jnp.bfloat16), w2_ref[0:128, :],
                preferred_element_type=jnp.float32)
    for j in range(1, 4):
        p += jnp.dot(a_ref[j].astype(jnp.bfloat16),
                     w2_ref[j * 128:(j + 1) * 128, :],
                     preferred_element_type=jnp.float32)
    h = jnp.maximum(_dinv(deg_ref) * p + b2_ref[...], 0.0)
    part = jnp.sum(h * wl_ref[...], axis=(0, 1), keepdims=True)

    @pl.when(i == 0)
    def _first():
        out_ref[...] = part

    @pl.when(i != 0)
    def _accum():
        out_ref[...] = out_ref[...] + part

    @pl.when(i == _NB - 1)
    def _final():
        v = out_ref[...] * (1.0 / N_NODES) + bl_ref[...]
        out_ref[...] = jax.nn.sigmoid(v)


_scale = pl.pallas_call(
    _scale_body,
    grid=(_NB,),
    in_specs=[
        pl.BlockSpec((_BN, D_IN), lambda i: (i, 0)),
        pl.BlockSpec((_BN, 2), lambda i: (i, 0)),
    ],
    out_specs=pl.BlockSpec((2, _BN, 128), lambda i: (0, i, 0)),
    out_shape=jax.ShapeDtypeStruct((2, N_NODES, 128), jnp.float32),
)

_mm1 = pl.pallas_call(
    _mm1_body,
    grid=(_NB,),
    in_specs=[
        pl.BlockSpec((2, _BN, 128), lambda i: (0, i, 0)),
        pl.BlockSpec((_BN, 2), lambda i: (i, 0)),
        pl.BlockSpec((D_IN, D_HID), lambda i: (0, 0)),
        pl.BlockSpec((1, D_HID), lambda i: (0, 0)),
    ],
    out_specs=pl.BlockSpec((4, _BN, 128), lambda i: (0, i, 0)),
    out_shape=jax.ShapeDtypeStruct((4, N_NODES, 128), jnp.float32),
)

_mm2 = pl.pallas_call(
    _mm2_body,
    grid=(_NB,),
    in_specs=[
        pl.BlockSpec((4, _BN, 128), lambda i: (0, i, 0)),
        pl.BlockSpec((_BN, 2), lambda i: (i, 0)),
        pl.BlockSpec((D_HID, D_HID), lambda i: (0, 0)),
        pl.BlockSpec((1, D_HID), lambda i: (0, 0)),
        pl.BlockSpec((1, D_HID), lambda i: (0, 0)),
        pl.BlockSpec((1, 1), lambda i: (0, 0)),
    ],
    out_specs=pl.BlockSpec((1, 1), lambda i: (0, 0)),
    out_shape=jax.ShapeDtypeStruct((1, 1), jnp.float32),
)


def kernel(x, edge_index, W1, b1, W2, b2, Wl, bl):
    src = edge_index[0]
    dst = edge_index[1]
    zeros_n = jnp.zeros((N_NODES,), jnp.float32)

    deg_parts = _deg_kernel(dst, zeros_n)
    deg_t = deg_parts.T

    g0 = _scale(x, deg_t)
    acc0 = _agg2(src, dst, g0.reshape(2 * N_NODES, 128))

    g1 = _mm1(acc0.reshape(2, N_NODES, 128), deg_t, W1.astype(jnp.bfloat16),
              b1.reshape(1, D_HID))
    acc1 = _agg4(src, dst, g1.reshape(4 * N_NODES, 128))

    out = _mm2(acc1.reshape(4, N_NODES, 128), deg_t, W2.astype(jnp.bfloat16),
               b2.reshape(1, D_HID), Wl.reshape(1, D_HID),
               bl.reshape(1, 1))
    return out.reshape(1)
